# Initial kernel scaffold; baseline (speedup 1.0000x reference)
import jax
import jax.numpy as jnp
from jax.experimental import pallas as pl


def kernel(alignment, shifts, coords, values, ctf):
    raise NotImplementedError("write your pallas kernel here")


# speedup vs baseline: 66.7969x; 66.7969x over previous
import functools

import jax
import jax.numpy as jnp
import numpy as np
from jax import lax
from jax.experimental import pallas as pl
from jax.experimental.pallas import tpu as pltpu
from jax.experimental.pallas import tpu_sc as plsc

XSIZE = 128
N_POINTS = 100000
BATCH = 64
SIGMA = 1.0

_NC = 2
_NS = 16
_NW = _NC * _NS
_IMGS_PER_W = BATCH // _NW
_XX = XSIZE * XSIZE
_CHUNK = 4000
_NCHUNK = N_POINTS // _CHUNK


def _sc_scatter_body(params_hbm, xs_hbm, ys_hbm, zs_hbm, vs_hbm, out_hbm,
                     params_v, xbuf, ybuf, zbuf, vbuf, img0, img1):
    wid = lax.axis_index("c") * _NS + lax.axis_index("s")

    pltpu.sync_copy(params_hbm.at[wid], params_v)

    zero16 = jnp.zeros((16,), jnp.float32)

    def zero_body(i, _):
        img0[pl.ds(i * 16, 16)] = zero16
        img1[pl.ds(i * 16, 16)] = zero16
        return _

    lax.fori_loop(0, _XX // 16, zero_body, None)

    p = [[params_v[s, j] for j in range(8)] for s in range(_IMGS_PER_W)]

    def chunk_body(c, _):
        base = c * _CHUNK
        pltpu.sync_copy(xs_hbm.at[pl.ds(base, _CHUNK)], xbuf)
        pltpu.sync_copy(ys_hbm.at[pl.ds(base, _CHUNK)], ybuf)
        pltpu.sync_copy(zs_hbm.at[pl.ds(base, _CHUNK)], zbuf)
        pltpu.sync_copy(vs_hbm.at[pl.ds(base, _CHUNK)], vbuf)

        def inner(i, _):
            off = i * 16
            xv = xbuf[pl.ds(off, 16)]
            yv = ybuf[pl.ds(off, 16)]
            zv = zbuf[pl.ds(off, 16)]
            vv = vbuf[pl.ds(off, 16)]
            for s in range(_IMGS_PER_W):
                img = img0 if s == 0 else img1
                px = xv * p[s][0] + yv * p[s][1] + zv * p[s][2] + p[s][6]
                py = xv * p[s][3] + yv * p[s][4] + zv * p[s][5] + p[s][7]
                tx = px.astype(jnp.int32)
                txf = tx.astype(jnp.float32)
                x0 = jnp.where(txf > px, tx - 1, tx)
                ty = py.astype(jnp.int32)
                tyf = ty.astype(jnp.float32)
                y0 = jnp.where(tyf > py, ty - 1, ty)
                fx = px - x0.astype(jnp.float32)
                fy = py - y0.astype(jnp.float32)
                ix = jnp.minimum(jnp.maximum(x0, 0), XSIZE - 2)
                iy = jnp.minimum(jnp.maximum(y0, 0), XSIZE - 2)
                idx = iy * XSIZE + ix
                vfy = vv * fy
                vw0 = vv - vfy
                vw1 = vfy
                a01 = vw0 * fx
                a11 = vw1 * fx
                a00 = vw0 - a01
                a10 = vw1 - a11
                plsc.addupdate_scatter(img, [idx], a00)
                plsc.addupdate_scatter(img, [idx + 1], a01)
                plsc.addupdate_scatter(img, [idx + XSIZE], a10)
                plsc.addupdate_scatter(img, [idx + (XSIZE + 1)], a11)
            return _

        lax.fori_loop(0, _CHUNK // 16, inner, None)
        return _

    lax.fori_loop(0, _NCHUNK, chunk_body, None)

    pltpu.sync_copy(img0, out_hbm.at[wid * _IMGS_PER_W])
    pltpu.sync_copy(img1, out_hbm.at[wid * _IMGS_PER_W + 1])


_sc_scatter = functools.partial(
    pl.kernel,
    mesh=plsc.VectorSubcoreMesh(core_axis_name="c", subcore_axis_name="s"),
    compiler_params=pltpu.CompilerParams(needs_layout_passes=False),
    out_type=jax.ShapeDtypeStruct((BATCH, _XX), jnp.float32),
    scratch_types=[
        pltpu.VMEM((_IMGS_PER_W, 8, 16), jnp.float32),
        pltpu.VMEM((_CHUNK,), jnp.float32),
        pltpu.VMEM((_CHUNK,), jnp.float32),
        pltpu.VMEM((_CHUNK,), jnp.float32),
        pltpu.VMEM((_CHUNK,), jnp.float32),
        pltpu.VMEM((_XX,), jnp.float32),
        pltpu.VMEM((_XX,), jnp.float32),
    ],
)(_sc_scatter_body)


def _filter_body(img_ref, h_ref, c_ref, s_ref, out_ref):
    mraw = img_ref[0]
    h = h_ref[0]
    cm = c_ref[...]
    sm = s_ref[...]
    mu = jnp.sum(mraw) * (1.0 / (XSIZE * XSIZE))
    m = mraw - mu

    def dot(a, b):
        return lax.dot(a, b, precision=lax.Precision.HIGHEST,
                       preferred_element_type=jnp.float32)

    cmm = dot(cm, m)
    smm = dot(sm, m)
    yr = dot(cmm, cm) - dot(smm, sm)
    yi = -(dot(smm, cm) + dot(cmm, sm))
    zr = yr * h
    zi = yi * h
    pr = dot(cm, zr) - dot(sm, zi)
    qi = dot(cm, zi) + dot(sm, zr)
    out_ref[0] = ((dot(pr, cm) - dot(qi, sm)) * (1.0 / (XSIZE * XSIZE))
                  + mu * h[0, 0])


def _make_dft_mats():
    k = np.arange(XSIZE)
    ang = 2.0 * np.pi * np.outer(k, k) / XSIZE
    return (jnp.asarray(np.cos(ang), dtype=jnp.float32),
            jnp.asarray(np.sin(ang), dtype=jnp.float32))


def kernel(alignment, shifts, coords, values, ctf):
    B, X = BATCH, XSIZE
    eps = 1e-8
    a1 = alignment[:, :3]
    a2 = alignment[:, 3:]
    b1 = a1 / (jnp.linalg.norm(a1, axis=1, keepdims=True) + eps)
    a2p = a2 - jnp.sum(b1 * a2, axis=1, keepdims=True) * b1
    b2 = a2p / (jnp.linalg.norm(a2p, axis=1, keepdims=True) + eps)
    params8 = jnp.concatenate(
        [b1, b2, shifts[:, 0:1] + X / 2.0, shifts[:, 1:2] + X / 2.0], axis=1)
    params = jnp.broadcast_to(
        params8.reshape(_NW, _IMGS_PER_W, 8, 1), (_NW, _IMGS_PER_W, 8, 16)
    ).astype(jnp.float32)

    xs = coords[:, 0]
    ys = coords[:, 1]
    zs = coords[:, 2]

    imgs = _sc_scatter(params, xs, ys, zs, values)
    imgs3 = imgs.reshape(B, X, X)

    fyv = np.fft.fftfreq(X).astype(np.float32)
    fxv = np.fft.rfftfreq(X).astype(np.float32)
    r2 = fyv[:, None] ** 2 + fxv[None, :] ** 2
    gauss = jnp.asarray(
        np.exp(-2.0 * (np.pi ** 2) * (SIGMA ** 2) * r2), dtype=jnp.float32)
    hh = gauss[None, :, :] * ctf
    flip_y = (-np.arange(X)) % X
    hfull = jnp.concatenate([hh, hh[:, flip_y][:, :, 63:0:-1]], axis=2)

    cmat, smat = _make_dft_mats()
    out = pl.pallas_call(
        _filter_body,
        grid=(B,),
        in_specs=[
            pl.BlockSpec((1, X, X), lambda b: (b, 0, 0)),
            pl.BlockSpec((1, X, X), lambda b: (b, 0, 0)),
            pl.BlockSpec((X, X), lambda b: (0, 0)),
            pl.BlockSpec((X, X), lambda b: (0, 0)),
        ],
        out_specs=pl.BlockSpec((1, X, X), lambda b: (b, 0, 0)),
        out_shape=jax.ShapeDtypeStruct((B, X, X), jnp.float32),
    )(imgs3, hfull, cmat, smat)
    return out


# speedup vs baseline: 84.7253x; 1.2684x over previous
import functools

import jax
import jax.numpy as jnp
import numpy as np
from jax import lax
from jax.experimental import pallas as pl
from jax.experimental.pallas import tpu as pltpu
from jax.experimental.pallas import tpu_sc as plsc

XSIZE = 128
N_POINTS = 100000
BATCH = 64
SIGMA = 1.0

_NC = 2
_NS = 16
_NW = _NC * _NS
_IMGS_PER_W = BATCH // _NW
_XX = XSIZE * XSIZE
_CHUNK = 2000
_NCHUNK = N_POINTS // _CHUNK
_BIAS = 512


def _sc_scatter_body(params_hbm, pts_hbm, out_hbm,
                     params_v, buf0, buf1, img0, img1, sem0, sem1):
    wid = lax.axis_index("c") * _NS + lax.axis_index("s")

    pltpu.sync_copy(params_hbm.at[wid], params_v)

    pltpu.async_copy(pts_hbm.at[0], buf0, sem0)

    zero16 = jnp.zeros((16,), jnp.float32)

    def zero_body(i, _):
        img0[pl.ds(i * 16, 16)] = zero16
        img1[pl.ds(i * 16, 16)] = zero16
        return _

    lax.fori_loop(0, _XX // 16, zero_body, None)

    p = [[params_v[s, j] for j in range(8)] for s in range(_IMGS_PER_W)]
    imgs = (img0, img1)
    taps = [(img, img.at[pl.ds(XSIZE, _XX - XSIZE)]) for img in imgs]

    def process(buf):
        def inner(i, _):
            off = i * 16
            xv = buf[0, pl.ds(off, 16)]
            yv = buf[1, pl.ds(off, 16)]
            zv = buf[2, pl.ds(off, 16)]
            vv = buf[3, pl.ds(off, 16)]
            for s in range(_IMGS_PER_W):
                px = xv * p[s][0] + yv * p[s][1] + zv * p[s][2] + p[s][6]
                py = xv * p[s][3] + yv * p[s][4] + zv * p[s][5] + p[s][7]
                xb = px.astype(jnp.int32)
                yb = py.astype(jnp.int32)
                fx = px - xb.astype(jnp.float32)
                fy = py - yb.astype(jnp.float32)
                ix = jnp.minimum(jnp.maximum(xb - _BIAS, 0), XSIZE - 2)
                iy = jnp.minimum(jnp.maximum(yb - _BIAS, 0), XSIZE - 2)
                idx = jnp.left_shift(iy, 7) + ix
                idx1 = idx + 1
                vfy = vv * fy
                vw0 = vv - vfy
                a01 = vw0 * fx
                a11 = vfy * fx
                a00 = vw0 - a01
                a10 = vfy - a11
                plsc.addupdate_scatter(taps[s][0], [idx], a00)
                plsc.addupdate_scatter(taps[s][0], [idx1], a01)
                plsc.addupdate_scatter(taps[s][1], [idx], a10)
                plsc.addupdate_scatter(taps[s][1], [idx1], a11)
            return _

        lax.fori_loop(0, _CHUNK // 16, inner, None)

    def wait(buf, sem):
        pltpu.make_async_copy(pts_hbm.at[0], buf, sem).wait()

    def pair_body(h, _):
        c = 2 * h
        pltpu.async_copy(pts_hbm.at[c + 1], buf1, sem1)
        wait(buf0, sem0)
        process(buf0)

        @pl.when(c + 2 < _NCHUNK)
        def _start_next():
            pltpu.async_copy(pts_hbm.at[c + 2], buf0, sem0)

        wait(buf1, sem1)
        process(buf1)
        return _

    lax.fori_loop(0, _NCHUNK // 2, pair_body, None)

    pltpu.sync_copy(img0, out_hbm.at[wid * _IMGS_PER_W])
    pltpu.sync_copy(img1, out_hbm.at[wid * _IMGS_PER_W + 1])


_sc_scatter = functools.partial(
    pl.kernel,
    mesh=plsc.VectorSubcoreMesh(core_axis_name="c", subcore_axis_name="s"),
    compiler_params=pltpu.CompilerParams(needs_layout_passes=False),
    out_type=jax.ShapeDtypeStruct((BATCH, _XX), jnp.float32),
    scratch_types=[
        pltpu.VMEM((_IMGS_PER_W, 8, 16), jnp.float32),
        pltpu.VMEM((4, _CHUNK), jnp.float32),
        pltpu.VMEM((4, _CHUNK), jnp.float32),
        pltpu.VMEM((_XX,), jnp.float32),
        pltpu.VMEM((_XX,), jnp.float32),
        pltpu.SemaphoreType.DMA,
        pltpu.SemaphoreType.DMA,
    ],
)(_sc_scatter_body)


def _filter_body(img_ref, h_ref, c_ref, s_ref, out_ref):
    mraw = img_ref[0]
    h = h_ref[0]
    cm = c_ref[...]
    sm = s_ref[...]
    mu = jnp.sum(mraw) * (1.0 / (XSIZE * XSIZE))
    m = mraw - mu

    def dot(a, b):
        return lax.dot(a, b, precision=lax.Precision.HIGHEST,
                       preferred_element_type=jnp.float32)

    cmm = dot(cm, m)
    smm = dot(sm, m)
    yr = dot(cmm, cm) - dot(smm, sm)
    yi = -(dot(smm, cm) + dot(cmm, sm))
    zr = yr * h
    zi = yi * h
    pr = dot(cm, zr) - dot(sm, zi)
    qi = dot(cm, zi) + dot(sm, zr)
    out_ref[0] = ((dot(pr, cm) - dot(qi, sm)) * (1.0 / (XSIZE * XSIZE))
                  + mu * h[0, 0])


def _make_dft_mats():
    k = np.arange(XSIZE)
    ang = 2.0 * np.pi * np.outer(k, k) / XSIZE
    return (jnp.asarray(np.cos(ang), dtype=jnp.float32),
            jnp.asarray(np.sin(ang), dtype=jnp.float32))


def kernel(alignment, shifts, coords, values, ctf):
    B, X = BATCH, XSIZE
    eps = 1e-8
    a1 = alignment[:, :3]
    a2 = alignment[:, 3:]
    b1 = a1 / (jnp.linalg.norm(a1, axis=1, keepdims=True) + eps)
    a2p = a2 - jnp.sum(b1 * a2, axis=1, keepdims=True) * b1
    b2 = a2p / (jnp.linalg.norm(a2p, axis=1, keepdims=True) + eps)
    params8 = jnp.concatenate(
        [b1, b2, shifts[:, 0:1] + (X / 2.0 + _BIAS),
         shifts[:, 1:2] + (X / 2.0 + _BIAS)], axis=1)
    params = jnp.broadcast_to(
        params8.reshape(_NW, _IMGS_PER_W, 8, 1), (_NW, _IMGS_PER_W, 8, 16)
    ).astype(jnp.float32)

    pts = jnp.concatenate([coords.T, values[None, :]], axis=0)
    pts = pts.reshape(4, _NCHUNK, _CHUNK).swapaxes(0, 1)

    imgs = _sc_scatter(params, pts)
    imgs3 = imgs.reshape(B, X, X)

    fyv = np.fft.fftfreq(X).astype(np.float32)
    fxv = np.fft.rfftfreq(X).astype(np.float32)
    r2 = fyv[:, None] ** 2 + fxv[None, :] ** 2
    gauss = jnp.asarray(
        np.exp(-2.0 * (np.pi ** 2) * (SIGMA ** 2) * r2), dtype=jnp.float32)
    hh = gauss[None, :, :] * ctf
    flip_y = (-np.arange(X)) % X
    hfull = jnp.concatenate([hh, hh[:, flip_y][:, :, 63:0:-1]], axis=2)

    cmat, smat = _make_dft_mats()
    out = pl.pallas_call(
        _filter_body,
        grid=(B,),
        in_specs=[
            pl.BlockSpec((1, X, X), lambda b: (b, 0, 0)),
            pl.BlockSpec((1, X, X), lambda b: (b, 0, 0)),
            pl.BlockSpec((X, X), lambda b: (0, 0)),
            pl.BlockSpec((X, X), lambda b: (0, 0)),
        ],
        out_specs=pl.BlockSpec((1, X, X), lambda b: (b, 0, 0)),
        out_shape=jax.ShapeDtypeStruct((B, X, X), jnp.float32),
    )(imgs3, hfull, cmat, smat)
    return out
